# Initial kernel scaffold; baseline (speedup 1.0000x reference)
import jax
import jax.numpy as jnp
from jax.experimental import pallas as pl


def kernel(edge_feat, edge_index, W_e2l, W0, W11, W21, W12, W22, W13, W23):
    raise NotImplementedError("write your pallas kernel here")


# speedup vs baseline: 4.3208x; 4.3208x over previous
import functools

import jax
import jax.numpy as jnp
from jax import lax
from jax.experimental import pallas as pl
from jax.experimental.pallas import tpu as pltpu
from jax.experimental.pallas import tpu_sc as plsc

N = 10000
E = 320000
D = 128
DE = 16

NC = 2
NS = 16
CH = 128
NCHUNK = E // CH
CHUNK_PER_CORE = NCHUNK // NC
TRIPS = (CHUNK_PER_CORE + NS - 1) // NS
NFULL = N // CH
NTAIL = N - NFULL * CH


def _ceil_div(a, b):
    return (a + b - 1) // b


def _make_segsum(gather: bool):
    mesh = plsc.VectorSubcoreMesh(core_axis_name="c", subcore_axis_name="s",
                                  num_cores=NC, num_subcores=NS)

    @functools.partial(
        pl.kernel,
        out_type=jax.ShapeDtypeStruct((NC, N, D), jnp.float32),
        mesh=mesh,
        scratch_types=[
            pltpu.VMEM((CH,), jnp.int32),
            pltpu.VMEM((CH,), jnp.int32),
            pltpu.VMEM((CH, D), jnp.float32),
            pltpu.VMEM_SHARED((N, D), jnp.float32),
            pltpu.SemaphoreType.DMA,
        ],
    )
    def segsum(vals_hbm, src_hbm, dst_hbm, zeros_hbm, out_hbm,
               sidx, didx, rows, acc, sem):
        c = lax.axis_index("c")
        s = lax.axis_index("s")

        for t in range(_ceil_div(NFULL, NS)):
            j = s + NS * t
            @pl.when(j < NFULL)
            def _():
                r0 = pl.multiple_of(j * CH, CH)
                pltpu.sync_copy(zeros_hbm, acc.at[pl.ds(r0, CH)])
        @pl.when(s == 0)
        def _():
            pltpu.sync_copy(zeros_hbm.at[pl.ds(0, NTAIL)],
                            acc.at[pl.ds(NFULL * CH, NTAIL)])
        plsc.subcore_barrier()

        def step(t, carry):
            local = t * NS + s
            @pl.when(local < CHUNK_PER_CORE)
            def _():
                base = (c * CHUNK_PER_CORE + local) * CH
                pltpu.sync_copy(dst_hbm.at[pl.ds(base, CH)], didx)
                if gather:
                    pltpu.sync_copy(src_hbm.at[pl.ds(base, CH)], sidx)
                    pltpu.async_copy(vals_hbm.at[sidx], rows, sem).wait()
                else:
                    pltpu.sync_copy(vals_hbm.at[pl.ds(base, CH)], rows)
                pltpu.sync_copy(rows, acc.at[didx], add=True)
            return carry

        lax.fori_loop(0, TRIPS, step, 0)
        plsc.subcore_barrier()

        for t in range(_ceil_div(NFULL, NS)):
            j = s + NS * t
            @pl.when(j < NFULL)
            def _():
                r0 = pl.multiple_of(j * CH, CH)
                pltpu.sync_copy(acc.at[pl.ds(r0, CH)], rows)
                pltpu.sync_copy(rows, out_hbm.at[c, pl.ds(r0, CH)])
        @pl.when(s == 0)
        def _():
            r0 = NFULL * CH
            pltpu.sync_copy(acc.at[pl.ds(r0, NTAIL)], rows.at[pl.ds(0, NTAIL)])
            pltpu.sync_copy(rows.at[pl.ds(0, NTAIL)],
                            out_hbm.at[c, pl.ds(r0, NTAIL)])

    return segsum


_segsum_gather = _make_segsum(True)
_segsum_linear = _make_segsum(False)


_BE = 8000
_BR = 2000


def _edge_linear_body(x_ref, w_ref, o_ref):
    o_ref[...] = jax.nn.relu(
        jnp.dot(x_ref[...], w_ref[...], preferred_element_type=jnp.float32))


def _edge_linear(edge_feat, W_e2l):
    return pl.pallas_call(
        _edge_linear_body,
        grid=(E // _BE,),
        in_specs=[
            pl.BlockSpec((_BE, DE), lambda i: (i, 0)),
            pl.BlockSpec((DE, D), lambda i: (0, 0)),
        ],
        out_specs=pl.BlockSpec((_BE, D), lambda i: (i, 0)),
        out_shape=jax.ShapeDtypeStruct((E, D), jnp.float32),
    )(edge_feat, W_e2l)


def _combine0_body(p_ref, w_ref, static_ref, cur_ref):
    pool = p_ref[0] + p_ref[1]
    sm = jnp.dot(pool, w_ref[...], preferred_element_type=jnp.float32)
    static_ref[...] = sm
    cur_ref[...] = jax.nn.relu(sm)


def _combine0(p, W0):
    return pl.pallas_call(
        _combine0_body,
        grid=(N // _BR,),
        in_specs=[
            pl.BlockSpec((NC, _BR, D), lambda i: (0, i, 0)),
            pl.BlockSpec((D, D), lambda i: (0, 0)),
        ],
        out_specs=[
            pl.BlockSpec((_BR, D), lambda i: (i, 0)),
            pl.BlockSpec((_BR, D), lambda i: (i, 0)),
        ],
        out_shape=[
            jax.ShapeDtypeStruct((N, D), jnp.float32),
            jax.ShapeDtypeStruct((N, D), jnp.float32),
        ],
    )(p, W0)


def _level_body(cur_ref, p_ref, static_ref, w1_ref, w2_ref, o_ref):
    pool = p_ref[0] + p_ref[1]
    acc = static_ref[...]
    acc += jnp.dot(cur_ref[...], w1_ref[...], preferred_element_type=jnp.float32)
    acc += jnp.dot(pool, w2_ref[...], preferred_element_type=jnp.float32)
    o_ref[...] = jax.nn.relu(acc)


def _level(cur, p, static, W1, W2):
    return pl.pallas_call(
        _level_body,
        grid=(N // _BR,),
        in_specs=[
            pl.BlockSpec((_BR, D), lambda i: (i, 0)),
            pl.BlockSpec((NC, _BR, D), lambda i: (0, i, 0)),
            pl.BlockSpec((_BR, D), lambda i: (i, 0)),
            pl.BlockSpec((D, D), lambda i: (0, 0)),
            pl.BlockSpec((D, D), lambda i: (0, 0)),
        ],
        out_specs=pl.BlockSpec((_BR, D), lambda i: (i, 0)),
        out_shape=jax.ShapeDtypeStruct((N, D), jnp.float32),
    )(cur, p, static, W1, W2)


def kernel(edge_feat, edge_index, W_e2l, W0, W11, W21, W12, W22, W13, W23):
    src = edge_index[0]
    dst = edge_index[1]
    zeros = jnp.zeros((CH, D), jnp.float32)

    edge_lin = _edge_linear(edge_feat, W_e2l)
    p = _segsum_linear(edge_lin, src, dst, zeros)
    static, cur = _combine0(p, W0)
    for W1, W2 in ((W11, W21), (W12, W22), (W13, W23)):
        p = _segsum_gather(cur, src, dst, zeros)
        cur = _level(cur, p, static, W1, W2)
    return cur
